# baseline (speedup 1.0000x reference)
import functools

import jax
import jax.numpy as jnp
from jax import lax
from jax.experimental import pallas as pl
from jax.experimental.pallas import tpu as pltpu
from jax.experimental.pallas import tpu_sc as plsc

D = 32
L = 16
NC, NS = 2, 16
NW = NC * NS
B = 16384
BPW = B // NW
NSLAB = 4
SLAB = BPW // NSLAB
GROUPS = BPW // L

_mesh = plsc.VectorSubcoreMesh(core_axis_name="c", subcore_axis_name="s")


@functools.partial(
    pl.kernel,
    out_type=jax.ShapeDtypeStruct((B,), jnp.float32),
    mesh=_mesh,
    compiler_params=pltpu.CompilerParams(
        needs_layout_passes=False, use_tc_tiling_on_sc=False),
    scratch_types=[
        pltpu.VMEM((NSLAB, SLAB), jnp.int32),
        pltpu.VMEM((NSLAB, SLAB), jnp.int32),
        pltpu.VMEM((BPW, D), jnp.float32),
        pltpu.VMEM((BPW, D), jnp.float32),
        pltpu.VMEM((D, L), jnp.float32),
        pltpu.VMEM((L,), jnp.float32),
        pltpu.VMEM((BPW,), jnp.float32),
        pltpu.SemaphoreType.DMA,
    ],
)
def _nnmf_sc(gi_hbm, si_hbm, gt_hbm, st_hbm, w_hbm, b_hbm, out_hbm,
             gi_v, si_v, g_v, s_v, w_v, b_v, o_v, sem):
    wid = lax.axis_index("s") * NC + lax.axis_index("c")
    pltpu.sync_copy(gi_hbm.at[wid], gi_v)
    pltpu.sync_copy(si_hbm.at[wid], si_v)
    copies = []
    for j in range(NSLAB):
        copies.append(pltpu.async_copy(
            gt_hbm.at[gi_v.at[j]], g_v.at[pl.ds(j * SLAB, SLAB)], sem))
        copies.append(pltpu.async_copy(
            st_hbm.at[si_v.at[j]], s_v.at[pl.ds(j * SLAB, SLAB)], sem))
    pltpu.sync_copy(w_hbm, w_v)
    pltpu.sync_copy(b_hbm, b_v)
    for c in copies:
        c.wait()

    wvecs = [w_v[d] for d in range(D)]
    bvec = b_v[...]

    def group_body(g0, carry):
        rows = g0 * L + lax.iota(jnp.int32, L)
        acc = bvec
        for d in range(D):
            cols = jnp.full((L,), d, jnp.int32)
            gcol = plsc.load_gather(g_v, [rows, cols])
            scol = plsc.load_gather(s_v, [rows, cols])
            acc = acc + gcol * scol * wvecs[d]
        o_v[pl.ds(g0 * L, L)] = acc
        return carry

    lax.fori_loop(0, GROUPS, group_body, 0)
    pltpu.sync_copy(o_v, out_hbm.at[pl.ds(wid * BPW, BPW)])


def kernel(gene_indices, spot_indices, gene_table, spot_table, W, b):
    gi = gene_indices.astype(jnp.int32).reshape(NW, NSLAB, SLAB)
    si = spot_indices.astype(jnp.int32).reshape(NW, NSLAB, SLAB)
    wsplat = jnp.broadcast_to(W.astype(jnp.float32), (D, L))
    bsplat = jnp.broadcast_to(b.astype(jnp.float32), (L,))
    out = _nnmf_sc(gi, si, gene_table.astype(jnp.float32),
                   spot_table.astype(jnp.float32), wsplat, bsplat)
    return out.reshape(B, 1)


# speedup vs baseline: 4.0224x; 4.0224x over previous
import functools

import jax
import jax.numpy as jnp
from jax import lax
from jax.experimental import pallas as pl
from jax.experimental.pallas import tpu as pltpu
from jax.experimental.pallas import tpu_sc as plsc

D = 32
L = 16
NSC, NT = 2, 16
B = 16384
BPT = B // NT
PPS = D // NSC
NSLAB = BPT // 128
VG = 100001
VS = 1000001

_mesh = plsc.VectorSubcoreMesh(core_axis_name="c", subcore_axis_name="s")


@functools.partial(
    pl.kernel,
    out_type=jax.ShapeDtypeStruct((NSC, B), jnp.float32),
    mesh=_mesh,
    compiler_params=pltpu.CompilerParams(needs_layout_passes=False),
    scratch_types=[
        pltpu.VMEM_SHARED((VS,), jnp.float32),
        pltpu.VMEM_SHARED((VG,), jnp.float32),
        pltpu.VMEM((NSLAB, 128), jnp.int32),
        pltpu.VMEM((NSLAB, 128), jnp.int32),
        pltpu.VMEM((BPT,), jnp.float32),
        pltpu.VMEM((BPT,), jnp.float32),
        pltpu.VMEM((D, L), jnp.float32),
        pltpu.VMEM((L,), jnp.float32),
        pltpu.VMEM((BPT,), jnp.float32),
        pltpu.SemaphoreType.DMA,
    ],
)
def _nnmf_sc(gi_hbm, si_hbm, gt_hbm, st_hbm, w_hbm, b_hbm, out_hbm,
             sp_buf, gp_buf, gi_v, si_v, g_val, s_val, w_v, b_v, acc_v, sem):
    c = lax.axis_index("c")
    sid = lax.axis_index("s")
    pltpu.sync_copy(gi_hbm.at[sid], gi_v)
    pltpu.sync_copy(si_hbm.at[sid], si_v)
    pltpu.sync_copy(w_hbm, w_v)
    pltpu.sync_copy(b_hbm, b_v)

    bscale = jnp.where(c == 0, 1.0, 0.0).astype(jnp.float32)
    binit = b_v[...] * bscale
    def init_body(k, carry):
        acc_v[pl.ds(k * L, L)] = binit
        return carry
    lax.fori_loop(0, BPT // L, init_body, 0)

    def plane_body(dd, carry):
        d = c * PPS + dd
        q = d // 8
        s = d % 8
        @pl.when(sid == 0)
        def _():
            pltpu.sync_copy(st_hbm.at[q, s], sp_buf)

        @pl.when(sid == 1)
        def _():
            pltpu.sync_copy(gt_hbm.at[q, s], gp_buf)

        plsc.subcore_barrier()
        copies = []
        for j in range(NSLAB):
            copies.append(pltpu.async_copy(
                sp_buf.at[si_v.at[j]], s_val.at[pl.ds(j * 128, 128)], sem))
            copies.append(pltpu.async_copy(
                gp_buf.at[gi_v.at[j]], g_val.at[pl.ds(j * 128, 128)], sem))
        for cp in copies:
            cp.wait()
        wvec = w_v[d]
        def fma_body(k, carry):
            i0 = k * L
            acc_v[pl.ds(i0, L)] = (acc_v[pl.ds(i0, L)]
                                   + g_val[pl.ds(i0, L)]
                                   * s_val[pl.ds(i0, L)] * wvec)
            return carry
        lax.fori_loop(0, BPT // L, fma_body, 0)
        plsc.subcore_barrier()
        return carry

    lax.fori_loop(0, PPS, plane_body, 0)
    pltpu.sync_copy(acc_v, out_hbm.at[c].at[pl.ds(sid * BPT, BPT)])


def kernel(gene_indices, spot_indices, gene_table, spot_table, W, b):
    gi = gene_indices.astype(jnp.int32).reshape(NT, NSLAB, 128)
    si = spot_indices.astype(jnp.int32).reshape(NT, NSLAB, 128)
    gt = jnp.transpose(gene_table.astype(jnp.float32)).reshape(D // 8, 8, VG)
    st = jnp.transpose(spot_table.astype(jnp.float32)).reshape(D // 8, 8, VS)
    wsplat = jnp.broadcast_to(W.astype(jnp.float32), (D, L))
    bsplat = jnp.broadcast_to(b.astype(jnp.float32), (L,))
    partials = _nnmf_sc(gi, si, gt, st, wsplat, bsplat)
    return (partials[0] + partials[1]).reshape(B, 1)


# speedup vs baseline: 4.0938x; 1.0178x over previous
import functools

import jax
import jax.numpy as jnp
from jax import lax
from jax.experimental import pallas as pl
from jax.experimental.pallas import tpu as pltpu
from jax.experimental.pallas import tpu_sc as plsc

D = 32
L = 16
NSC, NT = 2, 16
B = 16384
BPT = B // NT
PPS = D // NSC
NSLAB = BPT // 128
VG = 100001
VS = 1000001

_mesh = plsc.VectorSubcoreMesh(core_axis_name="c", subcore_axis_name="s")


@functools.partial(
    pl.kernel,
    out_type=jax.ShapeDtypeStruct((NSC, B), jnp.float32),
    mesh=_mesh,
    compiler_params=pltpu.CompilerParams(needs_layout_passes=False),
    scratch_types=[
        pltpu.VMEM_SHARED((VS,), jnp.float32),
        pltpu.VMEM_SHARED((VG,), jnp.float32),
        pltpu.VMEM((NSLAB, 128), jnp.int32),
        pltpu.VMEM((NSLAB, 128), jnp.int32),
        pltpu.VMEM((BPT,), jnp.float32),
        pltpu.VMEM((BPT,), jnp.float32),
        pltpu.VMEM((D, L), jnp.float32),
        pltpu.VMEM((L,), jnp.float32),
        pltpu.VMEM((BPT,), jnp.float32),
        pltpu.SemaphoreType.DMA,
    ],
)
def _nnmf_sc(gi_hbm, si_hbm, gt_hbm, st_hbm, w_hbm, b_hbm, out_hbm,
             sp_buf, gp_buf, gi_v, si_v, g_val, s_val, w_v, b_v, acc_v, sem):
    c = lax.axis_index("c")
    sid = lax.axis_index("s")
    pltpu.sync_copy(gi_hbm.at[sid], gi_v)
    pltpu.sync_copy(si_hbm.at[sid], si_v)
    pltpu.sync_copy(w_hbm, w_v)
    pltpu.sync_copy(b_hbm, b_v)

    bscale = jnp.where(c == 0, 1.0, 0.0).astype(jnp.float32)
    binit = b_v[...] * bscale
    def init_body(k, carry):
        acc_v[pl.ds(k * L, L)] = binit
        return carry
    lax.fori_loop(0, BPT // L, init_body, 0)

    def plane_body(dd, carry):
        d = c * PPS + dd
        q = d // 8
        s = d % 8
        @pl.when(sid == 0)
        def _():
            pltpu.sync_copy(st_hbm.at[q, s], sp_buf)

        @pl.when(sid == 1)
        def _():
            pltpu.sync_copy(gt_hbm.at[q, s], gp_buf)

        plsc.subcore_barrier()
        copies = []
        for j in range(NSLAB):
            copies.append(pltpu.async_copy(
                sp_buf.at[si_v.at[j]], s_val.at[pl.ds(j * 128, 128)], sem))
            copies.append(pltpu.async_copy(
                gp_buf.at[gi_v.at[j]], g_val.at[pl.ds(j * 128, 128)], sem))
        for cp in copies:
            cp.wait()
        plsc.subcore_barrier()
        wvec = w_v[d]
        def fma_body(k, carry):
            for u in range(8):
                i0 = k * (8 * L) + u * L
                acc_v[pl.ds(i0, L)] = (acc_v[pl.ds(i0, L)]
                                       + g_val[pl.ds(i0, L)]
                                       * s_val[pl.ds(i0, L)] * wvec)
            return carry
        lax.fori_loop(0, BPT // (8 * L), fma_body, 0)
        return carry

    lax.fori_loop(0, PPS, plane_body, 0)
    pltpu.sync_copy(acc_v, out_hbm.at[c].at[pl.ds(sid * BPT, BPT)])


def kernel(gene_indices, spot_indices, gene_table, spot_table, W, b):
    gi = gene_indices.astype(jnp.int32).reshape(NT, NSLAB, 128)
    si = spot_indices.astype(jnp.int32).reshape(NT, NSLAB, 128)
    gt = jnp.transpose(gene_table.astype(jnp.float32)).reshape(D // 8, 8, VG)
    st = jnp.transpose(spot_table.astype(jnp.float32)).reshape(D // 8, 8, VS)
    wsplat = jnp.broadcast_to(W.astype(jnp.float32), (D, L))
    bsplat = jnp.broadcast_to(b.astype(jnp.float32), (L,))
    partials = _nnmf_sc(gi, si, gt, st, wsplat, bsplat)
    return (partials[0] + partials[1]).reshape(B, 1)
